# Initial kernel scaffold; baseline (speedup 1.0000x reference)
import jax
import jax.numpy as jnp
from jax.experimental import pallas as pl


def kernel(v, faces):
    raise NotImplementedError("write your pallas kernel here")


# speedup vs baseline: 12.2858x; 12.2858x over previous
import functools

import jax
import jax.numpy as jnp
from jax import lax
from jax.experimental import pallas as pl
from jax.experimental.pallas import tpu as pltpu
from jax.experimental.pallas import tpu_sc as plsc

_B = 32
_N_VERTS = 100000
_N_FACES = 200000
_NW = 32
_CHUNK = 128
_K = 50
_F_PAD = _NW * _K * _CHUNK
_D = _B * 3
_NSL = _D // 16


def _edge_kernel(table_hbm, fidx_hbm, out_hbm, idx_v, rows_v, acc_v, sem):
    wid = lax.axis_index("s") * 2 + lax.axis_index("c")

    def chunk_body(k, accs):
        pltpu.sync_copy(fidx_hbm.at[wid, k], idx_v)
        h = []
        for j in range(3):
            h.append(pltpu.async_copy(
                table_hbm.at[idx_v.at[j]],
                rows_v.at[pl.ds(j * _CHUNK, _CHUNK)], sem))
        for j in range(3):
            h[j].wait()

        def face_body(f, accs):
            r = 3 * f
            out = []
            for s in range(_NSL):
                sl = pl.ds(s * 16, 16)
                a = rows_v[r, sl]
                b = rows_v[r + 1, sl]
                c = rows_v[r + 2, sl]
                t = jnp.abs(b - a) + jnp.abs(c - a) + jnp.abs(b - c)
                out.append(accs[s] + t)
            return tuple(out)

        return lax.fori_loop(0, _CHUNK, face_body, accs)

    zero = jnp.zeros((16,), jnp.float32)
    accs = lax.fori_loop(0, _K, chunk_body, (zero,) * _NSL)
    for s in range(_NSL):
        acc_v[pl.ds(s * 16, 16)] = accs[s]
    pltpu.sync_copy(acc_v, out_hbm.at[wid])


@jax.jit
def _edge_loss(table, fidx):
    mesh = plsc.VectorSubcoreMesh(core_axis_name="c", subcore_axis_name="s")
    partial = pl.kernel(
        _edge_kernel,
        mesh=mesh,
        out_type=jax.ShapeDtypeStruct((_NW, _D), jnp.float32),
        scratch_types=[
            pltpu.VMEM((3, _CHUNK), jnp.int32),
            pltpu.VMEM((3 * _CHUNK, _D), jnp.float32),
            pltpu.VMEM((_D,), jnp.float32),
            pltpu.SemaphoreType.DMA,
        ],
        compiler_params=pltpu.CompilerParams(use_tc_tiling_on_sc=False),
    )(table, fidx)
    return jnp.sum(partial) / (_B * _N_FACES * 3)


def kernel(v, faces):
    table = jnp.transpose(v, (1, 0, 2)).reshape(_N_VERTS, _D)
    f = faces.astype(jnp.int32)
    f = jnp.concatenate(
        [f, jnp.zeros((_F_PAD - _N_FACES, 3), jnp.int32)]).reshape(
            _NW, _K, 3, _CHUNK)
    return _edge_loss(table, f)
